# Initial kernel scaffold; baseline (speedup 1.0000x reference)
import jax
import jax.numpy as jnp
from jax.experimental import pallas as pl


def kernel(ids, adj, emb, prep_W, prep_b, a1_Wx, a1_Wn, a2_Wx, a2_Wn, fc_W, fc_b):
    raise NotImplementedError("write your pallas kernel here")


# speedup vs baseline: 2.6176x; 2.6176x over previous
import functools

import jax
import jax.numpy as jnp
from jax import lax
from jax.experimental import pallas as pl
from jax.experimental.pallas import tpu as pltpu
from jax.experimental.pallas import tpu_sc as plsc


def _sc_gather(ids, adj, emb):
    B = ids.shape[0]
    DEG = adj.shape[1]
    D = emb.shape[1]
    info = plsc.get_sparse_core_info()
    NW = info.num_cores * info.num_subcores
    IPW = B // NW
    C1 = IPW * DEG
    LCHUNK = 128
    NCH = C1 // LCHUNK
    mesh = plsc.VectorSubcoreMesh(core_axis_name="c", subcore_axis_name="s")

    @functools.partial(
        pl.kernel,
        mesh=mesh,
        compiler_params=pltpu.CompilerParams(use_tc_tiling_on_sc=False),
        out_type=(
            jax.ShapeDtypeStruct((B * DEG, D), jnp.float32),
            jax.ShapeDtypeStruct((B * DEG, D), jnp.float32),
        ),
        scratch_types=[
            pltpu.VMEM((IPW,), jnp.int32),
            pltpu.VMEM((IPW, DEG), jnp.int32),
            pltpu.VMEM((C1,), jnp.int32),
            pltpu.VMEM((C1, D), jnp.float32),
            pltpu.VMEM((C1, DEG), jnp.int32),
            pltpu.VMEM((C1, D), jnp.float32),
            pltpu.VMEM((DEG, D), jnp.float32),
            pltpu.SemaphoreType.DMA,
        ],
    )
    def body(ids_hbm, adj_hbm, emb_hbm, f1_out, m2_out,
             ids_v, n1_v, idx1_v, f1buf, n2buf, m2buf, gbuf, sem):
        wid = lax.axis_index("c") * info.num_subcores + lax.axis_index("s")
        base = wid * IPW
        pltpu.sync_copy(ids_hbm.at[pl.ds(base, IPW)], ids_v)
        pltpu.async_copy(adj_hbm.at[ids_v], n1_v, sem).wait()
        for i in range(IPW):
            idx1_v[pl.ds(i * DEG, DEG)] = n1_v[i, :]
        for c in range(NCH):
            idx = idx1_v.at[pl.ds(c * LCHUNK, LCHUNK)]
            pltpu.async_copy(emb_hbm.at[idx],
                             f1buf.at[pl.ds(c * LCHUNK, LCHUNK)], sem).wait()
            pltpu.async_copy(adj_hbm.at[idx],
                             n2buf.at[pl.ds(c * LCHUNK, LCHUNK)], sem).wait()
        pltpu.sync_copy(f1buf, f1_out.at[pl.ds(wid * C1, C1)])

        def j_body(j, carry):
            pltpu.async_copy(emb_hbm.at[n2buf.at[j]], gbuf, sem).wait()
            for d in range(D // 16):
                sl = pl.ds(d * 16, 16)
                acc = gbuf[0, sl]
                for r in range(1, DEG):
                    acc = acc + gbuf[r, sl]
                m2buf[j, sl] = acc
            return carry

        lax.fori_loop(0, C1, j_body, 0)
        pltpu.sync_copy(m2buf, m2_out.at[pl.ds(wid * C1, C1)])

    return body(ids, adj, emb)


def _tc_main(f1raw, m2sum, prep_W, prep_b, a1_Wx, a1_Wn, a2_Wn, deg):
    N, D = f1raw.shape
    R = 2048
    NB = N // R
    G = R // deg
    H = a1_Wx.shape[1]

    def body(f1_ref, m2_ref, pW, pb, wx, wn, w2n, hn_ref, m1_ref):
        dot = functools.partial(jnp.dot, precision=lax.Precision.HIGHEST)
        f1r = f1_ref[...]
        f1 = dot(f1r, pW[...]) + pb[...]
        m2 = dot(m2_ref[...] * (1.0 / deg), pW[...]) + pb[...]
        h1 = jnp.concatenate([dot(f1, wx[...]), dot(m2, wn[...])], axis=1)
        h1 = jnp.maximum(h1, 0.0)
        h1m = jnp.mean(h1.reshape(G, deg, 2 * H), axis=1)
        hn_ref[...] = dot(h1m, w2n[...])
        m1_ref[...] = jnp.mean(f1r.reshape(G, deg, D), axis=1)

    return pl.pallas_call(
        body,
        grid=(NB,),
        in_specs=[
            pl.BlockSpec((R, D), lambda i: (i, 0)),
            pl.BlockSpec((R, D), lambda i: (i, 0)),
            pl.BlockSpec((D, D), lambda i: (0, 0)),
            pl.BlockSpec((1, D), lambda i: (0, 0)),
            pl.BlockSpec((D, H), lambda i: (0, 0)),
            pl.BlockSpec((D, H), lambda i: (0, 0)),
            pl.BlockSpec((2 * H, H), lambda i: (0, 0)),
        ],
        out_specs=[
            pl.BlockSpec((G, H), lambda i: (i, 0)),
            pl.BlockSpec((G, D), lambda i: (i, 0)),
        ],
        out_shape=[
            jax.ShapeDtypeStruct((N // deg, H), jnp.float32),
            jax.ShapeDtypeStruct((N // deg, D), jnp.float32),
        ],
    )(f1raw, m2sum, prep_W, prep_b.reshape(1, D), a1_Wx, a1_Wn, a2_Wn)


def _tc_head(hn, m1raw, x0row, prep_W, prep_b, a1_Wx, a1_Wn, a2_Wx, fc_W, fc_b):
    B, H = hn.shape
    D = m1raw.shape[1]

    def body(hn_ref, m1_ref, x0_ref, pW, pb, wx, wn, w2x, fw, fb, out_ref):
        dot = functools.partial(jnp.dot, precision=lax.Precision.HIGHEST)
        m1 = dot(m1_ref[...], pW[...]) + pb[...]
        x0 = dot(x0_ref[...], pW[...]) + pb[...]
        xl = jnp.broadcast_to(dot(x0, wx[...]), (B, H))
        h0 = jnp.concatenate([xl, dot(m1, wn[...])], axis=1)
        h0 = jnp.maximum(h0, 0.0)
        g = jnp.concatenate([dot(h0, w2x[...]), hn_ref[...]], axis=1)
        nrm = jnp.maximum(jnp.sqrt(jnp.sum(g * g, axis=1, keepdims=True)), 1e-12)
        out_ref[...] = dot(g / nrm, fw[...]) + fb[...]

    return pl.pallas_call(
        body,
        out_shape=jax.ShapeDtypeStruct((B, 1), jnp.float32),
    )(hn, m1raw, x0row, prep_W, prep_b.reshape(1, D), a1_Wx, a1_Wn, a2_Wx,
      fc_W, fc_b.reshape(1, 1))


def kernel(ids, adj, emb, prep_W, prep_b, a1_Wx, a1_Wn, a2_Wx, a2_Wn, fc_W, fc_b):
    ids = ids.astype(jnp.int32)
    adj = adj.astype(jnp.int32)
    f1raw, m2sum = _sc_gather(ids, adj, emb)
    hn, m1raw = _tc_main(f1raw, m2sum, prep_W, prep_b, a1_Wx, a1_Wn, a2_Wn,
                         adj.shape[1])
    x0row = lax.slice(emb, (emb.shape[0] - 1, 0), (emb.shape[0], emb.shape[1]))
    return _tc_head(hn, m1raw, x0row, prep_W, prep_b, a1_Wx, a1_Wn, a2_Wx,
                    fc_W, fc_b)


# speedup vs baseline: 5.2520x; 2.0064x over previous
import functools

import jax
import jax.numpy as jnp
from jax import lax
from jax.experimental import pallas as pl
from jax.experimental.pallas import tpu as pltpu
from jax.experimental.pallas import tpu_sc as plsc


def _sc_gather(ids, adj, emb):
    B = ids.shape[0]
    DEG = adj.shape[1]
    D = emb.shape[1]
    info = plsc.get_sparse_core_info()
    NW = info.num_cores * info.num_subcores
    IPW = B // NW
    C1 = IPW * DEG
    LCHUNK = 128
    NCH = C1 // LCHUNK
    C2 = C1 * DEG
    NCH2 = C2 // LCHUNK
    NPAIR = NCH2 // 2
    GPC = LCHUNK // DEG
    mesh = plsc.VectorSubcoreMesh(core_axis_name="c", subcore_axis_name="s")

    @functools.partial(
        pl.kernel,
        mesh=mesh,
        compiler_params=pltpu.CompilerParams(use_tc_tiling_on_sc=False),
        out_type=(
            jax.ShapeDtypeStruct((B * DEG, D), jnp.float32),
            jax.ShapeDtypeStruct((B * DEG, D), jnp.float32),
        ),
        scratch_types=[
            pltpu.VMEM((IPW,), jnp.int32),
            pltpu.VMEM((IPW, DEG), jnp.int32),
            pltpu.VMEM((C1,), jnp.int32),
            pltpu.VMEM((C1, D), jnp.float32),
            pltpu.VMEM((C1, DEG), jnp.int32),
            pltpu.VMEM((C2,), jnp.int32),
            pltpu.VMEM((C1, D), jnp.float32),
            pltpu.VMEM((LCHUNK, D), jnp.float32),
            pltpu.VMEM((LCHUNK, D), jnp.float32),
            pltpu.SemaphoreType.DMA,
            pltpu.SemaphoreType.DMA,
            pltpu.SemaphoreType.DMA,
            pltpu.SemaphoreType.DMA,
        ],
    )
    def body(ids_hbm, adj_hbm, emb_hbm, f1_out, m2_out,
             ids_v, n1_v, idx1_v, f1buf, n2buf, idx2_v, m2buf,
             gbufA, gbufB, semH, semA, semB, semO):
        wid = lax.axis_index("c") * info.num_subcores + lax.axis_index("s")
        base = wid * IPW
        pltpu.sync_copy(ids_hbm.at[pl.ds(base, IPW)], ids_v)
        pltpu.async_copy(adj_hbm.at[ids_v], n1_v, semH).wait()
        for i in range(IPW):
            idx1_v[pl.ds(i * DEG, DEG)] = n1_v[i, :]
        for c in range(NCH):
            idx = idx1_v.at[pl.ds(c * LCHUNK, LCHUNK)]
            pltpu.async_copy(emb_hbm.at[idx],
                             f1buf.at[pl.ds(c * LCHUNK, LCHUNK)], semH)
            pltpu.async_copy(adj_hbm.at[idx],
                             n2buf.at[pl.ds(c * LCHUNK, LCHUNK)], semH)
        for c in range(NCH):
            pltpu.make_async_copy(
                emb_hbm.at[idx1_v.at[pl.ds(0, LCHUNK)]],
                f1buf.at[pl.ds(c * LCHUNK, LCHUNK)], semH).wait()
            pltpu.make_async_copy(
                adj_hbm.at[idx1_v.at[pl.ds(0, LCHUNK)]],
                n2buf.at[pl.ds(c * LCHUNK, LCHUNK)], semH).wait()
        f1_wb = pltpu.make_async_copy(f1buf, f1_out.at[pl.ds(wid * C1, C1)],
                                      semO)
        f1_wb.start()

        def fl_body(j, carry):
            for jj in range(4):
                r = j * 4 + jj
                idx2_v[pl.ds(r * DEG, DEG)] = n2buf[r, :]
            return carry
        lax.fori_loop(0, C1 // 4, fl_body, 0)

        def start2(c, buf, sem):
            pltpu.async_copy(
                emb_hbm.at[idx2_v.at[pl.ds(c * LCHUNK, LCHUNK)]], buf, sem)

        def wait2(buf, sem):
            pltpu.make_async_copy(
                emb_hbm.at[idx2_v.at[pl.ds(0, LCHUNK)]], buf, sem).wait()

        def accum(buf, row_base):
            for g in range(GPC):
                for d in range(D // 16):
                    sl = pl.ds(d * 16, 16)
                    acc = buf[g * DEG, sl]
                    for r in range(1, DEG):
                        acc = acc + buf[g * DEG + r, sl]
                    m2buf[row_base + g, sl] = acc

        start2(0, gbufA, semA)

        def t_body(t, carry):
            start2(2 * t + 1, gbufB, semB)
            wait2(gbufA, semA)
            accum(gbufA, 2 * t * GPC)

            @pl.when(t < NPAIR - 1)
            def _():
                start2(2 * t + 2, gbufA, semA)

            wait2(gbufB, semB)
            accum(gbufB, (2 * t + 1) * GPC)
            return carry

        lax.fori_loop(0, NPAIR, t_body, 0)
        pltpu.sync_copy(m2buf, m2_out.at[pl.ds(wid * C1, C1)])
        f1_wb.wait()

    return body(ids, adj, emb)


def _tc_main(f1raw, m2sum, prep_W, prep_b, a1_Wx, a1_Wn, a2_Wn, deg):
    N, D = f1raw.shape
    R = 2048
    NB = N // R
    G = R // deg
    H = a1_Wx.shape[1]

    def body(f1_ref, m2_ref, pW, pb, wx, wn, w2n, hn_ref, m1_ref):
        dot = functools.partial(jnp.dot, precision=lax.Precision.HIGHEST)
        f1r = f1_ref[...]
        f1 = dot(f1r, pW[...]) + pb[...]
        m2 = dot(m2_ref[...] * (1.0 / deg), pW[...]) + pb[...]
        h1 = jnp.concatenate([dot(f1, wx[...]), dot(m2, wn[...])], axis=1)
        h1 = jnp.maximum(h1, 0.0)
        h1m = jnp.mean(h1.reshape(G, deg, 2 * H), axis=1)
        hn_ref[...] = dot(h1m, w2n[...])
        m1_ref[...] = jnp.mean(f1r.reshape(G, deg, D), axis=1)

    return pl.pallas_call(
        body,
        grid=(NB,),
        in_specs=[
            pl.BlockSpec((R, D), lambda i: (i, 0)),
            pl.BlockSpec((R, D), lambda i: (i, 0)),
            pl.BlockSpec((D, D), lambda i: (0, 0)),
            pl.BlockSpec((1, D), lambda i: (0, 0)),
            pl.BlockSpec((D, H), lambda i: (0, 0)),
            pl.BlockSpec((D, H), lambda i: (0, 0)),
            pl.BlockSpec((2 * H, H), lambda i: (0, 0)),
        ],
        out_specs=[
            pl.BlockSpec((G, H), lambda i: (i, 0)),
            pl.BlockSpec((G, D), lambda i: (i, 0)),
        ],
        out_shape=[
            jax.ShapeDtypeStruct((N // deg, H), jnp.float32),
            jax.ShapeDtypeStruct((N // deg, D), jnp.float32),
        ],
    )(f1raw, m2sum, prep_W, prep_b.reshape(1, D), a1_Wx, a1_Wn, a2_Wn)


def _tc_head(hn, m1raw, x0row, prep_W, prep_b, a1_Wx, a1_Wn, a2_Wx, fc_W, fc_b):
    B, H = hn.shape
    D = m1raw.shape[1]

    def body(hn_ref, m1_ref, x0_ref, pW, pb, wx, wn, w2x, fw, fb, out_ref):
        dot = functools.partial(jnp.dot, precision=lax.Precision.HIGHEST)
        m1 = dot(m1_ref[...], pW[...]) + pb[...]
        x0 = dot(x0_ref[...], pW[...]) + pb[...]
        xl = jnp.broadcast_to(dot(x0, wx[...]), (B, H))
        h0 = jnp.concatenate([xl, dot(m1, wn[...])], axis=1)
        h0 = jnp.maximum(h0, 0.0)
        g = jnp.concatenate([dot(h0, w2x[...]), hn_ref[...]], axis=1)
        nrm = jnp.maximum(jnp.sqrt(jnp.sum(g * g, axis=1, keepdims=True)), 1e-12)
        out_ref[...] = dot(g / nrm, fw[...]) + fb[...]

    return pl.pallas_call(
        body,
        out_shape=jax.ShapeDtypeStruct((B, 1), jnp.float32),
    )(hn, m1raw, x0row, prep_W, prep_b.reshape(1, D), a1_Wx, a1_Wn, a2_Wx,
      fc_W, fc_b.reshape(1, 1))


def kernel(ids, adj, emb, prep_W, prep_b, a1_Wx, a1_Wn, a2_Wx, a2_Wn, fc_W, fc_b):
    ids = ids.astype(jnp.int32)
    adj = adj.astype(jnp.int32)
    f1raw, m2sum = _sc_gather(ids, adj, emb)
    hn, m1raw = _tc_main(f1raw, m2sum, prep_W, prep_b, a1_Wx, a1_Wn, a2_Wn,
                         adj.shape[1])
    x0row = lax.slice(emb, (emb.shape[0] - 1, 0), (emb.shape[0], emb.shape[1]))
    return _tc_head(hn, m1raw, x0row, prep_W, prep_b, a1_Wx, a1_Wn, a2_Wx,
                    fc_W, fc_b)
